# Initial kernel scaffold; baseline (speedup 1.0000x reference)
import jax
import jax.numpy as jnp
from jax.experimental import pallas as pl


def kernel(logits, prop_adj, struct_feat):
    raise NotImplementedError("write your pallas kernel here")


# speedup vs baseline: 1.2549x; 1.2549x over previous
import functools

import jax
import jax.numpy as jnp
import numpy as np
from jax.experimental import pallas as pl
from jax.experimental.pallas import tpu as pltpu

N = 10000
C = 16
PROP_STEPS = 2
ALPHA = 0.2
GLOBAL_BETA = 0.05
MIN_ANCHOR = 0.6
RESIDUAL_SCALE = 0.15
SOURCE_CONF_CENTER = 0.55
SOURCE_CONF_SHARPNESS = 8.0
RECIPIENT_CONF_CENTER = 0.5
RECIPIENT_CONF_SHARPNESS = 8.0
ACCEPT_SHARPNESS = 12.0
ACCEPT_QUALITY_WEIGHT = 0.7
ACCEPT_MARGIN_WEIGHT = 0.2
ACCEPT_STRUCT_WEIGHT = 0.1
EPS = 1e-8
MAX_ENTROPY = float(np.log(C))

K = 24
BM = 200


def _prologue_kernel(logits_ref, struct_ref, seed_ref, rhs_ref, aux_ref, gp_ref):
    logits = logits_ref[...]
    seed = jnp.maximum(logits, 0.0)
    score_mass = jnp.sum(seed, axis=1, keepdims=True)
    norm_scores = seed / (score_mass + EPS)
    entropy = -jnp.sum(norm_scores * jnp.log(norm_scores + EPS), axis=1,
                       keepdims=True)
    certainty = 1.0 - entropy / MAX_ENTROPY
    mass_scale = jnp.maximum(jnp.mean(score_mass), EPS)
    magnitude = jnp.tanh(score_mass / mass_scale)
    confidence = jnp.clip(0.5 * certainty + 0.5 * magnitude, 0.0, 1.0)

    weighted_seed = confidence * seed
    gp = (jnp.sum(weighted_seed, axis=0, keepdims=True)
          / jnp.maximum(jnp.sum(confidence), EPS))

    anchor = jnp.clip(MIN_ANCHOR + ALPHA * confidence, 0.0, 0.995)
    uncertainty = 1.0 - confidence
    clustering = struct_ref[...][:, 1:2]
    graph_scale = jnp.clip(1.0 - jnp.mean(clustering), 0.2, 1.0)
    source_gate = jax.nn.sigmoid(
        SOURCE_CONF_SHARPNESS * (confidence - SOURCE_CONF_CENTER))
    recipient_gate = jax.nn.sigmoid(
        RECIPIENT_CONF_SHARPNESS * (RECIPIENT_CONF_CENTER - confidence))

    seed_ref[...] = seed
    zeros_pad = jnp.zeros((seed.shape[0], K - C - 1), dtype=jnp.float32)
    rhs_ref[...] = jnp.concatenate([source_gate * seed, source_gate, zeros_pad],
                                   axis=1)
    gs_col = jnp.full_like(confidence, graph_scale)
    aux_ref[...] = jnp.concatenate(
        [recipient_gate, anchor, uncertainty, clustering, source_gate, gs_col,
         jnp.zeros((seed.shape[0], 2), dtype=jnp.float32)], axis=1)
    gp_ref[...] = gp


def _step_kernel(a_ref, rhs_ref, prop_ref, seed_ref, aux_ref, gp_ref,
                 prop_out_ref, rhs_out_ref):
    mm = jnp.dot(a_ref[...], rhs_ref[...], preferred_element_type=jnp.float32)
    num = mm[:, :C]
    den = jnp.maximum(mm[:, C:C + 1], EPS)
    local_context = num / den

    p = prop_ref[...]
    seed = seed_ref[...]
    aux = aux_ref[...]
    recipient_gate = aux[:, 0:1]
    anchor = aux[:, 1:2]
    uncertainty = aux[:, 2:3]
    clustering = aux[:, 3:4]
    source_gate = aux[:, 4:5]
    graph_scale = aux[:, 5:6]

    dotp = jnp.sum(p * local_context, axis=1, keepdims=True)
    na = jnp.maximum(jnp.sqrt(jnp.sum(p * p, axis=1, keepdims=True)), EPS)
    nb = jnp.maximum(
        jnp.sqrt(jnp.sum(local_context * local_context, axis=1, keepdims=True)),
        EPS)
    lq = jnp.clip((dotp / (na * nb) + 1.0) * 0.5, 0.0, 1.0)

    probs = p / (jnp.sum(p, axis=1, keepdims=True) + EPS)
    m1 = jnp.max(probs, axis=1, keepdims=True)
    am = jnp.argmax(probs, axis=1)[:, None]
    iota = jax.lax.broadcasted_iota(jnp.int32, probs.shape, 1)
    m2 = jnp.max(jnp.where(iota == am, -1.0, probs), axis=1, keepdims=True)
    margin = m1 - m2

    quality = (ACCEPT_QUALITY_WEIGHT * lq + ACCEPT_MARGIN_WEIGHT * margin
               + ACCEPT_STRUCT_WEIGHT * clustering)
    accept = jax.nn.sigmoid(ACCEPT_SHARPNESS * quality) * recipient_gate

    blend = (1.0 - GLOBAL_BETA) * local_context + GLOBAL_BETA * gp_ref[...]
    candidate = anchor * seed + (1.0 - anchor) * blend
    p_new = (p + accept * graph_scale * (candidate - p)
             + RESIDUAL_SCALE * uncertainty * (seed - p))

    prop_out_ref[...] = p_new
    zeros_pad = jnp.zeros((p_new.shape[0], K - C - 1), dtype=jnp.float32)
    rhs_out_ref[...] = jnp.concatenate(
        [source_gate * p_new, source_gate, zeros_pad], axis=1)


@functools.partial(jax.jit, static_argnames=())
def kernel(logits, prop_adj, struct_feat):
    n = logits.shape[0]
    seed, rhs, aux, gp = pl.pallas_call(
        _prologue_kernel,
        out_shape=[
            jax.ShapeDtypeStruct((n, C), jnp.float32),
            jax.ShapeDtypeStruct((n, K), jnp.float32),
            jax.ShapeDtypeStruct((n, 8), jnp.float32),
            jax.ShapeDtypeStruct((1, C), jnp.float32),
        ],
    )(logits, struct_feat)

    grid = n // BM
    step_call = pl.pallas_call(
        _step_kernel,
        grid=(grid,),
        in_specs=[
            pl.BlockSpec((BM, n), lambda i: (i, 0)),
            pl.BlockSpec((n, K), lambda i: (0, 0)),
            pl.BlockSpec((BM, C), lambda i: (i, 0)),
            pl.BlockSpec((BM, C), lambda i: (i, 0)),
            pl.BlockSpec((BM, 8), lambda i: (i, 0)),
            pl.BlockSpec((1, C), lambda i: (0, 0)),
        ],
        out_specs=[
            pl.BlockSpec((BM, C), lambda i: (i, 0)),
            pl.BlockSpec((BM, K), lambda i: (i, 0)),
        ],
        out_shape=[
            jax.ShapeDtypeStruct((n, C), jnp.float32),
            jax.ShapeDtypeStruct((n, K), jnp.float32),
        ],
    )

    propagated = seed
    for _ in range(PROP_STEPS):
        propagated, rhs = step_call(prop_adj, rhs, propagated, seed, aux, gp)
    return propagated


# speedup vs baseline: 1.2788x; 1.0190x over previous
import functools

import jax
import jax.numpy as jnp
import numpy as np
from jax.experimental import pallas as pl
from jax.experimental.pallas import tpu as pltpu

N = 10000
C = 16
PROP_STEPS = 2
ALPHA = 0.2
GLOBAL_BETA = 0.05
MIN_ANCHOR = 0.6
RESIDUAL_SCALE = 0.15
SOURCE_CONF_CENTER = 0.55
SOURCE_CONF_SHARPNESS = 8.0
RECIPIENT_CONF_CENTER = 0.5
RECIPIENT_CONF_SHARPNESS = 8.0
ACCEPT_SHARPNESS = 12.0
ACCEPT_QUALITY_WEIGHT = 0.7
ACCEPT_MARGIN_WEIGHT = 0.2
ACCEPT_STRUCT_WEIGHT = 0.1
EPS = 1e-8
MAX_ENTROPY = float(np.log(C))

K = 24
BM = 400


def _prologue_kernel(logits_ref, struct_ref, seed_ref, rhs_ref, aux_ref, gp_ref):
    logits = logits_ref[...]
    seed = jnp.maximum(logits, 0.0)
    score_mass = jnp.sum(seed, axis=1, keepdims=True)
    norm_scores = seed / (score_mass + EPS)
    entropy = -jnp.sum(norm_scores * jnp.log(norm_scores + EPS), axis=1,
                       keepdims=True)
    certainty = 1.0 - entropy / MAX_ENTROPY
    mass_scale = jnp.maximum(jnp.mean(score_mass), EPS)
    magnitude = jnp.tanh(score_mass / mass_scale)
    confidence = jnp.clip(0.5 * certainty + 0.5 * magnitude, 0.0, 1.0)

    weighted_seed = confidence * seed
    gp = (jnp.sum(weighted_seed, axis=0, keepdims=True)
          / jnp.maximum(jnp.sum(confidence), EPS))

    anchor = jnp.clip(MIN_ANCHOR + ALPHA * confidence, 0.0, 0.995)
    uncertainty = 1.0 - confidence
    clustering = struct_ref[...][:, 1:2]
    graph_scale = jnp.clip(1.0 - jnp.mean(clustering), 0.2, 1.0)
    source_gate = jax.nn.sigmoid(
        SOURCE_CONF_SHARPNESS * (confidence - SOURCE_CONF_CENTER))
    recipient_gate = jax.nn.sigmoid(
        RECIPIENT_CONF_SHARPNESS * (RECIPIENT_CONF_CENTER - confidence))

    seed_ref[...] = seed
    zeros_pad = jnp.zeros((seed.shape[0], K - C - 1), dtype=jnp.float32)
    rhs_ref[...] = jnp.concatenate([source_gate * seed, source_gate, zeros_pad],
                                   axis=1)
    gs_col = jnp.full_like(confidence, graph_scale)
    aux_ref[...] = jnp.concatenate(
        [recipient_gate, anchor, uncertainty, clustering, source_gate, gs_col,
         jnp.zeros((seed.shape[0], 2), dtype=jnp.float32)], axis=1)
    gp_ref[...] = gp


def _step_kernel(a_ref, rhs_ref, prop_ref, seed_ref, aux_ref, gp_ref,
                 prop_out_ref, rhs_out_ref):
    mm = jnp.dot(a_ref[...], rhs_ref[...], preferred_element_type=jnp.float32)
    num = mm[:, :C]
    den = jnp.maximum(mm[:, C:C + 1], EPS)
    local_context = num / den

    p = prop_ref[...]
    seed = seed_ref[...]
    aux = aux_ref[...]
    recipient_gate = aux[:, 0:1]
    anchor = aux[:, 1:2]
    uncertainty = aux[:, 2:3]
    clustering = aux[:, 3:4]
    source_gate = aux[:, 4:5]
    graph_scale = aux[:, 5:6]

    dotp = jnp.sum(p * local_context, axis=1, keepdims=True)
    na = jnp.maximum(jnp.sqrt(jnp.sum(p * p, axis=1, keepdims=True)), EPS)
    nb = jnp.maximum(
        jnp.sqrt(jnp.sum(local_context * local_context, axis=1, keepdims=True)),
        EPS)
    lq = jnp.clip((dotp / (na * nb) + 1.0) * 0.5, 0.0, 1.0)

    probs = p / (jnp.sum(p, axis=1, keepdims=True) + EPS)
    m1 = jnp.max(probs, axis=1, keepdims=True)
    am = jnp.argmax(probs, axis=1)[:, None]
    iota = jax.lax.broadcasted_iota(jnp.int32, probs.shape, 1)
    m2 = jnp.max(jnp.where(iota == am, -1.0, probs), axis=1, keepdims=True)
    margin = m1 - m2

    quality = (ACCEPT_QUALITY_WEIGHT * lq + ACCEPT_MARGIN_WEIGHT * margin
               + ACCEPT_STRUCT_WEIGHT * clustering)
    accept = jax.nn.sigmoid(ACCEPT_SHARPNESS * quality) * recipient_gate

    blend = (1.0 - GLOBAL_BETA) * local_context + GLOBAL_BETA * gp_ref[...]
    candidate = anchor * seed + (1.0 - anchor) * blend
    p_new = (p + accept * graph_scale * (candidate - p)
             + RESIDUAL_SCALE * uncertainty * (seed - p))

    prop_out_ref[...] = p_new
    zeros_pad = jnp.zeros((p_new.shape[0], K - C - 1), dtype=jnp.float32)
    rhs_out_ref[...] = jnp.concatenate(
        [source_gate * p_new, source_gate, zeros_pad], axis=1)


@functools.partial(jax.jit, static_argnames=())
def kernel(logits, prop_adj, struct_feat):
    n = logits.shape[0]
    seed, rhs, aux, gp = pl.pallas_call(
        _prologue_kernel,
        out_shape=[
            jax.ShapeDtypeStruct((n, C), jnp.float32),
            jax.ShapeDtypeStruct((n, K), jnp.float32),
            jax.ShapeDtypeStruct((n, 8), jnp.float32),
            jax.ShapeDtypeStruct((1, C), jnp.float32),
        ],
    )(logits, struct_feat)

    grid = n // BM
    step_call = pl.pallas_call(
        _step_kernel,
        grid=(grid,),
        in_specs=[
            pl.BlockSpec((BM, n), lambda i: (i, 0)),
            pl.BlockSpec((n, K), lambda i: (0, 0)),
            pl.BlockSpec((BM, C), lambda i: (i, 0)),
            pl.BlockSpec((BM, C), lambda i: (i, 0)),
            pl.BlockSpec((BM, 8), lambda i: (i, 0)),
            pl.BlockSpec((1, C), lambda i: (0, 0)),
        ],
        out_specs=[
            pl.BlockSpec((BM, C), lambda i: (i, 0)),
            pl.BlockSpec((BM, K), lambda i: (i, 0)),
        ],
        out_shape=[
            jax.ShapeDtypeStruct((n, C), jnp.float32),
            jax.ShapeDtypeStruct((n, K), jnp.float32),
        ],
    )

    propagated = seed
    for _ in range(PROP_STEPS):
        propagated, rhs = step_call(prop_adj, rhs, propagated, seed, aux, gp)
    return propagated
